# baseline (speedup 1.0000x reference)
import functools

import jax
import jax.numpy as jnp
from jax import lax
from jax.experimental import pallas as pl
from jax.experimental.pallas import tpu as pltpu
from jax.experimental.pallas import tpu_sc as plsc

B = 256
H = 1024
N = 32640

SPLIT = 24576
TCR = 512
NBLK = SPLIT // TCR

NC = 2
NS = 16
L = 16
NW = NC * NS
SC_ROWS = N - SPLIT
ROWS_W = 256
SC_TAIL = SC_ROWS - (NW - 1) * ROWS_W
CH = 16
HL = H // L
QWIN = 64


def _qk_tc_body(ctx_ref, qw_ref, kw_ref, nl_ref, nlc_ref,
                qk_ref, qkbf_ref, end_ref, endc_ref, startc_ref):
    msq = lax.dot_general(ctx_ref[...], qw_ref[...],
                          (((1,), (1,)), ((), ())),
                          preferred_element_type=jnp.float32)
    qk = lax.dot_general(msq, kw_ref[...],
                         (((1,), (0,)), ((), ())),
                         preferred_element_type=jnp.float32)
    qk_ref[...] = qk
    qkbf_ref[...] = qk.astype(jnp.bfloat16)
    lens = nl_ref[...].astype(jnp.float32)
    ii = lax.broadcasted_iota(jnp.int32, (B, B), 0)
    jj = lax.broadcasted_iota(jnp.int32, (B, B), 1)
    tri = jnp.where(ii <= jj, 1.0, 0.0).astype(jnp.float32)
    endf = lax.dot_general(lens, tri, (((1,), (0,)), ((), ())),
                           preferred_element_type=jnp.float32)
    end_ref[...] = (endf + 0.5).astype(jnp.int32)

    lensc = nlc_ref[...].astype(jnp.float32)
    tril = jnp.where(jj <= ii, 1.0, 0.0).astype(jnp.float32)
    endcf = lax.dot_general(tril, lensc, (((1,), (0,)), ((), ())),
                            preferred_element_type=jnp.float32)
    endc = (endcf + 0.5).astype(jnp.int32)
    endc_ref[...] = endc
    startc_ref[...] = endc - nlc_ref[...]


def _qk_and_end(context, ms_q_w, ms_k_w, node_lengths):
    return pl.pallas_call(
        _qk_tc_body,
        out_shape=[
            jax.ShapeDtypeStruct((B, H), jnp.float32),
            jax.ShapeDtypeStruct((B, H), jnp.bfloat16),
            jax.ShapeDtypeStruct((1, B), jnp.int32),
            jax.ShapeDtypeStruct((B, 1), jnp.int32),
            jax.ShapeDtypeStruct((B, 1), jnp.int32),
        ],
    )(context, ms_q_w, ms_k_w,
      node_lengths.reshape(1, B), node_lengths.reshape(B, 1))


def _tcr_body(qk_ref, g_ref, endc_ref, startc_ref, mask_ref, out_ref):
    p = lax.dot_general(qk_ref[...], g_ref[...].astype(jnp.bfloat16),
                        (((1,), (1,)), ((), ())),
                        preferred_element_type=jnp.float32)
    rowbase = pl.program_id(0) * TCR
    cols = rowbase + lax.broadcasted_iota(jnp.int32, (B, TCR), 1)
    endb = jnp.broadcast_to(endc_ref[...], (B, TCR))
    startb = jnp.broadcast_to(startc_ref[...], (B, TCR))
    oh = (cols >= startb) & (cols < endb)
    vals = jnp.sum(jnp.where(oh, p, 0.0), axis=0, keepdims=True)
    out_ref[...] = jnp.where(mask_ref[...] != 0,
                             vals.reshape(1, 1, TCR), -jnp.inf)


def _tc_logits(qkbf, g_head, endc, startc, mask_head):
    return pl.pallas_call(
        _tcr_body,
        grid=(NBLK,),
        in_specs=[
            pl.BlockSpec((B, H), lambda i: (0, 0)),
            pl.BlockSpec((TCR, H), lambda i: (i, 0)),
            pl.BlockSpec((B, 1), lambda i: (0, 0)),
            pl.BlockSpec((B, 1), lambda i: (0, 0)),
            pl.BlockSpec((1, 1, TCR), lambda i: (i, 0, 0)),
        ],
        out_specs=pl.BlockSpec((1, 1, TCR), lambda i: (i, 0, 0)),
        out_shape=jax.ShapeDtypeStruct((NBLK, 1, TCR), jnp.float32),
    )(qkbf, g_head, endc, startc, mask_head)


def _sc_body(g_hbm, qk_hbm, end_hbm, mask_hbm, out_hbm,
             qk_win, gbuf, end_v, mask_v, out_v, sem_g):
    cid = lax.axis_index("c")
    sid = lax.axis_index("s")
    wid = sid * NC + cid
    r0l = wid * ROWS_W
    r0 = SPLIT + r0l

    pltpu.sync_copy(end_hbm, end_v)
    pltpu.sync_copy(mask_hbm.at[pl.ds(r0, ROWS_W)], mask_v)

    lane = lax.iota(jnp.int32, L)
    neg_inf = jnp.full((L,), -jnp.inf, jnp.float32)

    def _segment_of(rows):
        lo = jnp.zeros((L,), jnp.int32)
        w = B // 2
        while w >= 1:
            e = plsc.load_gather(end_v, [lo + (w - 1)])
            lo = lo + jnp.where(e <= rows, w, 0).astype(jnp.int32)
            w //= 2
        return lo

    s0 = _segment_of(jnp.full((L,), r0, jnp.int32))[0]
    sw = (jnp.minimum(s0, B - QWIN) // 8) * 8
    pltpu.sync_copy(qk_hbm.at[pl.ds(sw, QWIN)], qk_win)

    nchunk = jnp.minimum(SC_ROWS - r0l, ROWS_W) // CH

    def _g_copy(row_base, slot):
        return pltpu.make_async_copy(
            g_hbm.at[pl.ds(row_base, CH)],
            gbuf.at[pl.ds(slot * CH, CH)], sem_g)

    _g_copy(r0, 0).start()

    def _chunk(j, carry):
        slot = lax.rem(j, 2)
        row_base = r0 + j * CH
        _g_copy(row_base, slot).wait()

        @pl.when(j + 1 < nchunk)
        def _():
            _g_copy(row_base + CH, lax.rem(j + 1, 2)).start()

        rows = row_base + lane
        qrow = jnp.clip(_segment_of(rows) - sw, 0, QWIN - 1)

        def _row(r, vals):
            q = jnp.take_along_axis(qrow, jnp.full((L,), r, jnp.int32),
                                    axis=0)[0]
            gb = slot * CH + r
            acc = [gbuf[gb, pl.ds(p * L, L)] * qk_win[q, pl.ds(p * L, L)]
                   for p in range(8)]
            for h in range(8, HL):
                acc[h % 8] = acc[h % 8] + (gbuf[gb, pl.ds(h * L, L)] *
                                           qk_win[q, pl.ds(h * L, L)])
            tot = (((acc[0] + acc[1]) + (acc[2] + acc[3])) +
                   ((acc[4] + acc[5]) + (acc[6] + acc[7])))
            for sh in (8, 4, 2, 1):
                tot = tot + jnp.take_along_axis(tot, lane ^ sh, axis=0)
            return jnp.where(lane == r, tot, vals)
        vals = lax.fori_loop(0, CH, _row, jnp.zeros((L,), jnp.float32))

        mv = mask_v[pl.ds(j * CH, CH)]
        out_v[pl.ds(j * CH, CH)] = jnp.where(mv != 0, vals, neg_inf)
        return carry

    lax.fori_loop(0, nchunk, _chunk, 0)

    @pl.when(r0l + ROWS_W <= SC_ROWS)
    def _():
        pltpu.sync_copy(out_v, out_hbm.at[pl.ds(r0l, ROWS_W)])

    @pl.when(r0l + ROWS_W > SC_ROWS)
    def _():
        pltpu.sync_copy(out_v.at[pl.ds(0, SC_TAIL)],
                        out_hbm.at[pl.ds(r0l, SC_TAIL)])


@functools.lru_cache(maxsize=1)
def _sc_logits():
    return pl.kernel(
        _sc_body,
        out_type=jax.ShapeDtypeStruct((SC_ROWS,), jnp.float32),
        mesh=plsc.VectorSubcoreMesh(core_axis_name="c", subcore_axis_name="s",
                                    num_cores=NC, num_subcores=NS),
        compiler_params=pltpu.CompilerParams(needs_layout_passes=False),
        scratch_types=[
            pltpu.VMEM((QWIN, H), jnp.float32),
            pltpu.VMEM((2 * CH, H), jnp.float32),
            pltpu.VMEM((B,), jnp.int32),
            pltpu.VMEM((ROWS_W,), jnp.int32),
            pltpu.VMEM((ROWS_W,), jnp.float32),
            pltpu.SemaphoreType.DMA,
        ],
    )


def kernel(context, graph_embeds, machine_mask, node_lengths, ms_q_w, ms_k_w):
    qk, qkbf, end2d, endc, startc = _qk_and_end(
        context, ms_q_w, ms_k_w, node_lengths)
    end = end2d.reshape(B)
    mask_i32 = jnp.pad(machine_mask.astype(jnp.int32),
                       (0, SPLIT + NW * ROWS_W - N))
    sc_out = _sc_logits()(graph_embeds, qk, end, mask_i32)
    tc_out = _tc_logits(qkbf, graph_embeds, endc, startc,
                        mask_i32[:SPLIT].reshape(NBLK, 1, TCR))
    return jnp.concatenate([tc_out.reshape(SPLIT), sc_out])


# speedup vs baseline: 1.0441x; 1.0441x over previous
import functools

import jax
import jax.numpy as jnp
from jax import lax
from jax.experimental import pallas as pl
from jax.experimental.pallas import tpu as pltpu
from jax.experimental.pallas import tpu_sc as plsc

B = 256
H = 1024
N = 32640

SPLIT = 22528
TCR = 512
NBLK = SPLIT // TCR

NC = 2
NS = 16
L = 16
NW = NC * NS
SC_ROWS = N - SPLIT
ROWS_W = 320
SC_TAIL = SC_ROWS - (NW - 1) * ROWS_W
CH = 16
HL = H // L
QWIN = 64


def _qk_tc_body(ctx_ref, qw_ref, kw_ref, nl_ref, nlc_ref,
                qk_ref, qkbf_ref, end_ref, endc_ref, startc_ref):
    msq = lax.dot_general(ctx_ref[...], qw_ref[...],
                          (((1,), (1,)), ((), ())),
                          preferred_element_type=jnp.float32)
    qk = lax.dot_general(msq, kw_ref[...],
                         (((1,), (0,)), ((), ())),
                         preferred_element_type=jnp.float32)
    qk_ref[...] = qk
    qkbf_ref[...] = qk.astype(jnp.bfloat16)
    lens = nl_ref[...].astype(jnp.float32)
    ii = lax.broadcasted_iota(jnp.int32, (B, B), 0)
    jj = lax.broadcasted_iota(jnp.int32, (B, B), 1)
    tri = jnp.where(ii <= jj, 1.0, 0.0).astype(jnp.float32)
    endf = lax.dot_general(lens, tri, (((1,), (0,)), ((), ())),
                           preferred_element_type=jnp.float32)
    end_ref[...] = (endf + 0.5).astype(jnp.int32)

    lensc = nlc_ref[...].astype(jnp.float32)
    tril = jnp.where(jj <= ii, 1.0, 0.0).astype(jnp.float32)
    endcf = lax.dot_general(tril, lensc, (((1,), (0,)), ((), ())),
                            preferred_element_type=jnp.float32)
    endc = (endcf + 0.5).astype(jnp.int32)
    endc_ref[...] = endc
    startc_ref[...] = endc - nlc_ref[...]


def _qk_and_end(context, ms_q_w, ms_k_w, node_lengths):
    return pl.pallas_call(
        _qk_tc_body,
        out_shape=[
            jax.ShapeDtypeStruct((B, H), jnp.float32),
            jax.ShapeDtypeStruct((B, H), jnp.bfloat16),
            jax.ShapeDtypeStruct((1, B), jnp.int32),
            jax.ShapeDtypeStruct((B, 1), jnp.int32),
            jax.ShapeDtypeStruct((B, 1), jnp.int32),
        ],
    )(context, ms_q_w, ms_k_w,
      node_lengths.reshape(1, B), node_lengths.reshape(B, 1))


def _tcr_body(qk_ref, g_ref, endc_ref, startc_ref, mask_ref, out_ref):
    p = lax.dot_general(qk_ref[...], g_ref[...].astype(jnp.bfloat16),
                        (((1,), (1,)), ((), ())),
                        preferred_element_type=jnp.float32)
    rowbase = pl.program_id(0) * TCR
    cols = rowbase + lax.broadcasted_iota(jnp.int32, (B, TCR), 1)
    endb = jnp.broadcast_to(endc_ref[...], (B, TCR))
    startb = jnp.broadcast_to(startc_ref[...], (B, TCR))
    oh = (cols >= startb) & (cols < endb)
    vals = jnp.sum(jnp.where(oh, p, 0.0), axis=0, keepdims=True)
    out_ref[...] = jnp.where(mask_ref[...] != 0,
                             vals.reshape(1, 1, TCR), -jnp.inf)


def _tc_logits(qkbf, g_head, endc, startc, mask_head):
    return pl.pallas_call(
        _tcr_body,
        grid=(NBLK,),
        in_specs=[
            pl.BlockSpec((B, H), lambda i: (0, 0)),
            pl.BlockSpec((TCR, H), lambda i: (i, 0)),
            pl.BlockSpec((B, 1), lambda i: (0, 0)),
            pl.BlockSpec((B, 1), lambda i: (0, 0)),
            pl.BlockSpec((1, 1, TCR), lambda i: (i, 0, 0)),
        ],
        out_specs=pl.BlockSpec((1, 1, TCR), lambda i: (i, 0, 0)),
        out_shape=jax.ShapeDtypeStruct((NBLK, 1, TCR), jnp.float32),
    )(qkbf, g_head, endc, startc, mask_head)


def _sc_body(g_hbm, qk_hbm, end_hbm, mask_hbm, out_hbm,
             qk_win, gbuf, end_v, mask_v, out_v, sem_g):
    cid = lax.axis_index("c")
    sid = lax.axis_index("s")
    wid = sid * NC + cid
    r0l = wid * ROWS_W
    r0 = SPLIT + r0l

    pltpu.sync_copy(end_hbm, end_v)
    pltpu.sync_copy(mask_hbm.at[pl.ds(r0, ROWS_W)], mask_v)

    lane = lax.iota(jnp.int32, L)
    neg_inf = jnp.full((L,), -jnp.inf, jnp.float32)

    def _segment_of(rows):
        lo = jnp.zeros((L,), jnp.int32)
        w = B // 2
        while w >= 1:
            e = plsc.load_gather(end_v, [lo + (w - 1)])
            lo = lo + jnp.where(e <= rows, w, 0).astype(jnp.int32)
            w //= 2
        return lo

    s0 = _segment_of(jnp.full((L,), r0, jnp.int32))[0]
    sw = (jnp.minimum(s0, B - QWIN) // 8) * 8
    pltpu.sync_copy(qk_hbm.at[pl.ds(sw, QWIN)], qk_win)

    nchunk = jnp.minimum(SC_ROWS - r0l, ROWS_W) // CH

    def _g_copy(row_base, slot):
        return pltpu.make_async_copy(
            g_hbm.at[pl.ds(row_base, CH)],
            gbuf.at[pl.ds(slot * CH, CH)], sem_g)

    _g_copy(r0, 0).start()

    def _chunk(j, carry):
        slot = lax.rem(j, 2)
        row_base = r0 + j * CH
        _g_copy(row_base, slot).wait()

        @pl.when(j + 1 < nchunk)
        def _():
            _g_copy(row_base + CH, lax.rem(j + 1, 2)).start()

        rows = row_base + lane
        qrow = jnp.clip(_segment_of(rows) - sw, 0, QWIN - 1)

        def _row(r, vals):
            q = jnp.take_along_axis(qrow, jnp.full((L,), r, jnp.int32),
                                    axis=0)[0]
            gb = slot * CH + r
            acc = [gbuf[gb, pl.ds(p * L, L)] * qk_win[q, pl.ds(p * L, L)]
                   for p in range(8)]
            for h in range(8, HL):
                acc[h % 8] = acc[h % 8] + (gbuf[gb, pl.ds(h * L, L)] *
                                           qk_win[q, pl.ds(h * L, L)])
            tot = (((acc[0] + acc[1]) + (acc[2] + acc[3])) +
                   ((acc[4] + acc[5]) + (acc[6] + acc[7])))
            for sh in (8, 4, 2, 1):
                tot = tot + jnp.take_along_axis(tot, lane ^ sh, axis=0)
            return jnp.where(lane == r, tot, vals)
        vals = lax.fori_loop(0, CH, _row, jnp.zeros((L,), jnp.float32))

        mv = mask_v[pl.ds(j * CH, CH)]
        out_v[pl.ds(j * CH, CH)] = jnp.where(mv != 0, vals, neg_inf)
        return carry

    lax.fori_loop(0, nchunk, _chunk, 0)

    @pl.when(r0l + ROWS_W <= SC_ROWS)
    def _():
        pltpu.sync_copy(out_v, out_hbm.at[pl.ds(r0l, ROWS_W)])

    @pl.when(r0l + ROWS_W > SC_ROWS)
    def _():
        pltpu.sync_copy(out_v.at[pl.ds(0, SC_TAIL)],
                        out_hbm.at[pl.ds(r0l, SC_TAIL)])


@functools.lru_cache(maxsize=1)
def _sc_logits():
    return pl.kernel(
        _sc_body,
        out_type=jax.ShapeDtypeStruct((SC_ROWS,), jnp.float32),
        mesh=plsc.VectorSubcoreMesh(core_axis_name="c", subcore_axis_name="s",
                                    num_cores=NC, num_subcores=NS),
        compiler_params=pltpu.CompilerParams(needs_layout_passes=False),
        scratch_types=[
            pltpu.VMEM((QWIN, H), jnp.float32),
            pltpu.VMEM((2 * CH, H), jnp.float32),
            pltpu.VMEM((B,), jnp.int32),
            pltpu.VMEM((ROWS_W,), jnp.int32),
            pltpu.VMEM((ROWS_W,), jnp.float32),
            pltpu.SemaphoreType.DMA,
        ],
    )


def kernel(context, graph_embeds, machine_mask, node_lengths, ms_q_w, ms_k_w):
    qk, qkbf, end2d, endc, startc = _qk_and_end(
        context, ms_q_w, ms_k_w, node_lengths)
    end = end2d.reshape(B)
    mask_i32 = jnp.pad(machine_mask.astype(jnp.int32),
                       (0, SPLIT + NW * ROWS_W - N))
    sc_out = _sc_logits()(graph_embeds, qk, end, mask_i32)
    tc_out = _tc_logits(qkbf, graph_embeds, endc, startc,
                        mask_i32[:SPLIT].reshape(NBLK, 1, TCR))
    return jnp.concatenate([tc_out.reshape(SPLIT), sc_out])


# speedup vs baseline: 1.0676x; 1.0225x over previous
import functools

import jax
import jax.numpy as jnp
from jax import lax
from jax.experimental import pallas as pl
from jax.experimental.pallas import tpu as pltpu
from jax.experimental.pallas import tpu_sc as plsc

B = 256
H = 1024
N = 32640

SPLIT = 22528
TCR = 1024
NBLK = SPLIT // TCR

NC = 2
NS = 16
L = 16
NW = NC * NS
SC_ROWS = N - SPLIT
ROWS_W = 320
SC_TAIL = SC_ROWS - (NW - 1) * ROWS_W
CH = 16
HL = H // L
QWIN = 64


def _qk_tc_body(ctx_ref, qw_ref, kw_ref, nl_ref, nlc_ref,
                qk_ref, qkbf_ref, end_ref, endc_ref, startc_ref):
    msq = lax.dot_general(ctx_ref[...], qw_ref[...],
                          (((1,), (1,)), ((), ())),
                          preferred_element_type=jnp.float32)
    qk = lax.dot_general(msq, kw_ref[...],
                         (((1,), (0,)), ((), ())),
                         preferred_element_type=jnp.float32)
    qk_ref[...] = qk
    qkbf_ref[...] = qk.astype(jnp.bfloat16)
    lens = nl_ref[...].astype(jnp.float32)
    ii = lax.broadcasted_iota(jnp.int32, (B, B), 0)
    jj = lax.broadcasted_iota(jnp.int32, (B, B), 1)
    tri = jnp.where(ii <= jj, 1.0, 0.0).astype(jnp.float32)
    endf = lax.dot_general(lens, tri, (((1,), (0,)), ((), ())),
                           preferred_element_type=jnp.float32)
    end_ref[...] = (endf + 0.5).astype(jnp.int32)

    lensc = nlc_ref[...].astype(jnp.float32)
    tril = jnp.where(jj <= ii, 1.0, 0.0).astype(jnp.float32)
    endcf = lax.dot_general(tril, lensc, (((1,), (0,)), ((), ())),
                            preferred_element_type=jnp.float32)
    endc = (endcf + 0.5).astype(jnp.int32)
    endc_ref[...] = endc
    startc_ref[...] = endc - nlc_ref[...]


def _qk_and_end(context, ms_q_w, ms_k_w, node_lengths):
    return pl.pallas_call(
        _qk_tc_body,
        out_shape=[
            jax.ShapeDtypeStruct((B, H), jnp.float32),
            jax.ShapeDtypeStruct((B, H), jnp.bfloat16),
            jax.ShapeDtypeStruct((1, B), jnp.int32),
            jax.ShapeDtypeStruct((B, 1), jnp.int32),
            jax.ShapeDtypeStruct((B, 1), jnp.int32),
        ],
    )(context, ms_q_w, ms_k_w,
      node_lengths.reshape(1, B), node_lengths.reshape(B, 1))


def _tcr_body(qk_ref, g_ref, endc_ref, startc_ref, mask_ref, out_ref):
    p = lax.dot_general(qk_ref[...], g_ref[...].astype(jnp.bfloat16),
                        (((1,), (1,)), ((), ())),
                        preferred_element_type=jnp.float32)
    rowbase = pl.program_id(0) * TCR
    cols = rowbase + lax.broadcasted_iota(jnp.int32, (B, TCR), 1)
    endb = jnp.broadcast_to(endc_ref[...], (B, TCR))
    startb = jnp.broadcast_to(startc_ref[...], (B, TCR))
    oh = (cols >= startb) & (cols < endb)
    vals = jnp.sum(jnp.where(oh, p, 0.0), axis=0, keepdims=True)
    out_ref[...] = jnp.where(mask_ref[...] != 0,
                             vals.reshape(1, 1, TCR), -jnp.inf)


def _tc_logits(qkbf, g_head, endc, startc, mask_head):
    return pl.pallas_call(
        _tcr_body,
        grid=(NBLK,),
        in_specs=[
            pl.BlockSpec((B, H), lambda i: (0, 0)),
            pl.BlockSpec((TCR, H), lambda i: (i, 0)),
            pl.BlockSpec((B, 1), lambda i: (0, 0)),
            pl.BlockSpec((B, 1), lambda i: (0, 0)),
            pl.BlockSpec((1, 1, TCR), lambda i: (i, 0, 0)),
        ],
        out_specs=pl.BlockSpec((1, 1, TCR), lambda i: (i, 0, 0)),
        out_shape=jax.ShapeDtypeStruct((NBLK, 1, TCR), jnp.float32),
    )(qkbf, g_head, endc, startc, mask_head)


def _sc_body(g_hbm, qk_hbm, end_hbm, mask_hbm, out_hbm,
             qk_win, gbuf, end_v, mask_v, out_v, sem_g):
    cid = lax.axis_index("c")
    sid = lax.axis_index("s")
    wid = sid * NC + cid
    r0l = wid * ROWS_W
    r0 = SPLIT + r0l

    pltpu.sync_copy(end_hbm, end_v)
    pltpu.sync_copy(mask_hbm.at[pl.ds(r0, ROWS_W)], mask_v)

    lane = lax.iota(jnp.int32, L)
    neg_inf = jnp.full((L,), -jnp.inf, jnp.float32)

    def _segment_of(rows):
        lo = jnp.zeros((L,), jnp.int32)
        w = B // 2
        while w >= 1:
            e = plsc.load_gather(end_v, [lo + (w - 1)])
            lo = lo + jnp.where(e <= rows, w, 0).astype(jnp.int32)
            w //= 2
        return lo

    s0 = _segment_of(jnp.full((L,), r0, jnp.int32))[0]
    sw = (jnp.minimum(s0, B - QWIN) // 8) * 8
    pltpu.sync_copy(qk_hbm.at[pl.ds(sw, QWIN)], qk_win)

    nchunk = jnp.minimum(SC_ROWS - r0l, ROWS_W) // CH

    def _g_copy(row_base, slot):
        return pltpu.make_async_copy(
            g_hbm.at[pl.ds(row_base, CH)],
            gbuf.at[pl.ds(slot * CH, CH)], sem_g)

    _g_copy(r0, 0).start()

    def _chunk(j, carry):
        slot = lax.rem(j, 2)
        row_base = r0 + j * CH
        _g_copy(row_base, slot).wait()

        @pl.when(j + 1 < nchunk)
        def _():
            _g_copy(row_base + CH, lax.rem(j + 1, 2)).start()

        rows = row_base + lane
        qrow = jnp.clip(_segment_of(rows) - sw, 0, QWIN - 1)

        def _row(r, vals):
            q = jnp.take_along_axis(qrow, jnp.full((L,), r, jnp.int32),
                                    axis=0)[0]
            gb = slot * CH + r
            acc = [gbuf[gb, pl.ds(p * L, L)] * qk_win[q, pl.ds(p * L, L)]
                   for p in range(8)]
            for h in range(8, HL):
                acc[h % 8] = acc[h % 8] + (gbuf[gb, pl.ds(h * L, L)] *
                                           qk_win[q, pl.ds(h * L, L)])
            tot = (((acc[0] + acc[1]) + (acc[2] + acc[3])) +
                   ((acc[4] + acc[5]) + (acc[6] + acc[7])))
            for sh in (8, 4, 2, 1):
                tot = tot + jnp.take_along_axis(tot, lane ^ sh, axis=0)
            return jnp.where(lane == r, tot, vals)
        vals = lax.fori_loop(0, CH, _row, jnp.zeros((L,), jnp.float32))

        mv = mask_v[pl.ds(j * CH, CH)]
        out_v[pl.ds(j * CH, CH)] = jnp.where(mv != 0, vals, neg_inf)
        return carry

    lax.fori_loop(0, nchunk, _chunk, 0)

    @pl.when(r0l + ROWS_W <= SC_ROWS)
    def _():
        pltpu.sync_copy(out_v, out_hbm.at[pl.ds(r0l, ROWS_W)])

    @pl.when(r0l + ROWS_W > SC_ROWS)
    def _():
        pltpu.sync_copy(out_v.at[pl.ds(0, SC_TAIL)],
                        out_hbm.at[pl.ds(r0l, SC_TAIL)])


@functools.lru_cache(maxsize=1)
def _sc_logits():
    return pl.kernel(
        _sc_body,
        out_type=jax.ShapeDtypeStruct((SC_ROWS,), jnp.float32),
        mesh=plsc.VectorSubcoreMesh(core_axis_name="c", subcore_axis_name="s",
                                    num_cores=NC, num_subcores=NS),
        compiler_params=pltpu.CompilerParams(needs_layout_passes=False),
        scratch_types=[
            pltpu.VMEM((QWIN, H), jnp.float32),
            pltpu.VMEM((2 * CH, H), jnp.float32),
            pltpu.VMEM((B,), jnp.int32),
            pltpu.VMEM((ROWS_W,), jnp.int32),
            pltpu.VMEM((ROWS_W,), jnp.float32),
            pltpu.SemaphoreType.DMA,
        ],
    )


def kernel(context, graph_embeds, machine_mask, node_lengths, ms_q_w, ms_k_w):
    qk, qkbf, end2d, endc, startc = _qk_and_end(
        context, ms_q_w, ms_k_w, node_lengths)
    end = end2d.reshape(B)
    mask_i32 = jnp.pad(machine_mask.astype(jnp.int32),
                       (0, SPLIT + NW * ROWS_W - N))
    sc_out = _sc_logits()(graph_embeds, qk, end, mask_i32)
    tc_out = _tc_logits(qkbf, graph_embeds, endc, startc,
                        mask_i32[:SPLIT].reshape(NBLK, 1, TCR))
    return jnp.concatenate([tc_out.reshape(SPLIT), sc_out])
